# baseline (speedup 1.0000x reference)
import functools

import jax
import jax.numpy as jnp
from jax import lax
from jax.experimental import pallas as pl
from jax.experimental.pallas import tpu as pltpu
from jax.experimental.pallas import tpu_sc as plsc

_B = 16384
_D = 128

_SLICES = (8192, 4096, 4096)

_NC = 2
_NS = 16
_NW = _NC * _NS


@functools.cache
def _make_gather(bh, row_off):
    mesh = plsc.VectorSubcoreMesh(core_axis_name="c", subcore_axis_name="s")
    rpw = bh // _NW
    ch = rpw
    while ch > 256:
        ch //= 2
    n_chunks = rpw // ch

    @functools.partial(
        pl.kernel,
        mesh=mesh,
        out_type=[
            jax.ShapeDtypeStruct((bh, _D), jnp.float32),
            jax.ShapeDtypeStruct((bh, _D), jnp.float32),
        ],
        scratch_types=[
            pltpu.VMEM((rpw,), jnp.int32),
            pltpu.VMEM((rpw,), jnp.int32),
            pltpu.VMEM((ch, _D), jnp.float32),
            pltpu.VMEM((ch, _D), jnp.float32),
            pltpu.VMEM((ch, _D), jnp.float32),
            pltpu.SemaphoreType.DMA,
            pltpu.SemaphoreType.DMA,
            pltpu.SemaphoreType.DMA,
            pltpu.SemaphoreType.DMA,
        ],
    )
    def _gather2(uidx_hbm, iidx_hbm, table_hbm, out_x, out_y,
                 uix_v, iix_v, bufa, bufb, bufc, gsa, gsb, gsc, st_sem):
        wid = lax.axis_index("s") * _NC + lax.axis_index("c")
        base = wid * rpw
        pltpu.sync_copy(uidx_hbm.at[pl.ds(row_off + base, rpw)], uix_v)
        pltpu.sync_copy(iidx_hbm.at[pl.ds(row_off + base, rpw)], iix_v)
        sched = []
        for c in range(n_chunks):
            sched.append((uix_v, c * ch, out_x))
        for c in range(n_chunks):
            sched.append((iix_v, c * ch, out_y))
        bufs = [(bufa, gsa), (bufb, gsb), (bufc, gsc)]
        nb = len(bufs)
        gathers = []
        stores = []
        for k, (idx_v, off, _out) in enumerate(sched):
            buf, sem = bufs[k % nb]
            gathers.append(
                pltpu.async_copy(table_hbm.at[idx_v.at[pl.ds(off, ch)]], buf, sem)
            )
            if k >= nb - 1:
                pidx = k - (nb - 1)
                gathers[pidx].wait()
                pbuf, _ = bufs[pidx % nb]
                _, poff, pout = sched[pidx]
                stores.append(
                    pltpu.async_copy(pbuf, pout.at[pl.ds(base + poff, ch)], st_sem)
                )
        for pidx in range(max(0, len(sched) - (nb - 1)), len(sched)):
            gathers[pidx].wait()
            pbuf, _ = bufs[pidx % nb]
            _, poff, pout = sched[pidx]
            stores.append(
                pltpu.async_copy(pbuf, pout.at[pl.ds(base + poff, ch)], st_sem)
            )
        for st in stores:
            st.wait()

    return _gather2


_BS = 4096


def _mlp_block(x, y, w1_ref, b1, w2, b2, w3, b3, w4, b4):
    h = jnp.dot(x, w1_ref[0], preferred_element_type=jnp.float32)
    h = h + jnp.dot(y, w1_ref[1], preferred_element_type=jnp.float32)
    h = jnp.maximum(h + b1[...], 0.0)
    h = jnp.maximum(jnp.dot(h, w2[...], preferred_element_type=jnp.float32) + b2[...], 0.0)
    h = jnp.maximum(jnp.dot(h, w3[...], preferred_element_type=jnp.float32) + b3[...], 0.0)
    z = jax.lax.dot_general(
        w4[...], h, (((1,), (1,)), ((), ())),
        preferred_element_type=jnp.float32)[0] + b4[0]
    return 1.0 / (1.0 + jnp.exp(-z))


def _mlp_body(x_ref, y_ref, w1_ref, b1, w2, b2, w3, b3, w4, b4, o_ref):
    o_ref[...] = _mlp_block(x_ref[...], y_ref[...],
                            w1_ref, b1, w2, b2, w3, b3, w4, b4)


def _mlp_body_acc(prev_ref, x_ref, y_ref, w1_ref, b1, w2, b2, w3, b3, w4, b4,
                  o_ref):
    del prev_ref
    o_ref[...] = _mlp_block(x_ref[...], y_ref[...],
                            w1_ref, b1, w2, b2, w3, b3, w4, b4)


def _full(shape):
    return pl.BlockSpec(shape, lambda i: tuple(0 for _ in shape))


_W_SPECS = [
    _full((2, _D, 128)),
    _full((128,)),
    _full((128, 64)),
    _full((64,)),
    _full((64, 32)),
    _full((32,)),
    _full((1, 32)),
    _full((1,)),
]


def _mlp(blk_off, nblk, xg, yg, weights, prev=None):
    xy = pl.BlockSpec((_BS, _D), lambda i: (i, 0))
    out = pl.BlockSpec((_BS,), lambda i, o=blk_off: (o + i,))
    if prev is None:
        return pl.pallas_call(
            _mlp_body,
            grid=(nblk,),
            in_specs=[xy, xy] + _W_SPECS,
            out_specs=out,
            out_shape=jax.ShapeDtypeStruct((_B,), jnp.float32),
            compiler_params=pltpu.CompilerParams(
                dimension_semantics=("arbitrary",)),
        )(xg, yg, *weights)
    return pl.pallas_call(
        _mlp_body_acc,
        grid=(nblk,),
        in_specs=[pl.BlockSpec(memory_space=pl.ANY), xy, xy] + _W_SPECS,
        out_specs=out,
        out_shape=jax.ShapeDtypeStruct((_B,), jnp.float32),
        input_output_aliases={0: 0},
        compiler_params=pltpu.CompilerParams(
            dimension_semantics=("arbitrary",)),
    )(prev, xg, yg, *weights)


def kernel(user_input, item_input, user_emb, W1, b1, W2, b2, W3, b3, W4, b4):
    uidx = user_input.astype(jnp.int32)
    iidx = item_input.astype(jnp.int32)
    weights = (W1.reshape(2, _D, 128), b1, W2, b2, W3, b3, W4.reshape(1, 32), b4)
    out = None
    off = 0
    for bh in _SLICES:
        xg, yg = _make_gather(bh, off)(uidx, iidx, user_emb)
        out = _mlp(off // _BS, bh // _BS, xg, yg, weights, prev=out)
        off += bh
    return out


# speedup vs baseline: 1.0910x; 1.0910x over previous
import functools

import jax
import jax.numpy as jnp
from jax import lax
from jax.experimental import pallas as pl
from jax.experimental.pallas import tpu as pltpu
from jax.experimental.pallas import tpu_sc as plsc

_B = 16384
_D = 128

_SLICES = (12288, 4096)

_NC = 2
_NS = 16
_NW = _NC * _NS


@functools.cache
def _make_gather(bh, row_off):
    mesh = plsc.VectorSubcoreMesh(core_axis_name="c", subcore_axis_name="s")
    rpw = bh // _NW
    ch = rpw
    while ch > 256:
        ch //= 2
    n_chunks = rpw // ch

    @functools.partial(
        pl.kernel,
        mesh=mesh,
        out_type=[
            jax.ShapeDtypeStruct((bh, _D), jnp.float32),
            jax.ShapeDtypeStruct((bh, _D), jnp.float32),
        ],
        scratch_types=[
            pltpu.VMEM((rpw,), jnp.int32),
            pltpu.VMEM((rpw,), jnp.int32),
            pltpu.VMEM((ch, _D), jnp.float32),
            pltpu.VMEM((ch, _D), jnp.float32),
            pltpu.VMEM((ch, _D), jnp.float32),
            pltpu.SemaphoreType.DMA,
            pltpu.SemaphoreType.DMA,
            pltpu.SemaphoreType.DMA,
            pltpu.SemaphoreType.DMA,
        ],
    )
    def _gather2(uidx_hbm, iidx_hbm, table_hbm, out_x, out_y,
                 uix_v, iix_v, bufa, bufb, bufc, gsa, gsb, gsc, st_sem):
        wid = lax.axis_index("s") * _NC + lax.axis_index("c")
        base = wid * rpw
        pltpu.sync_copy(uidx_hbm.at[pl.ds(row_off + base, rpw)], uix_v)
        pltpu.sync_copy(iidx_hbm.at[pl.ds(row_off + base, rpw)], iix_v)
        sched = []
        for c in range(n_chunks):
            sched.append((uix_v, c * ch, out_x))
        for c in range(n_chunks):
            sched.append((iix_v, c * ch, out_y))
        bufs = [(bufa, gsa), (bufb, gsb), (bufc, gsc)]
        nb = len(bufs)
        gathers = []
        stores = []
        for k, (idx_v, off, _out) in enumerate(sched):
            buf, sem = bufs[k % nb]
            gathers.append(
                pltpu.async_copy(table_hbm.at[idx_v.at[pl.ds(off, ch)]], buf, sem)
            )
            if k >= nb - 1:
                pidx = k - (nb - 1)
                gathers[pidx].wait()
                pbuf, _ = bufs[pidx % nb]
                _, poff, pout = sched[pidx]
                stores.append(
                    pltpu.async_copy(pbuf, pout.at[pl.ds(base + poff, ch)], st_sem)
                )
        for pidx in range(max(0, len(sched) - (nb - 1)), len(sched)):
            gathers[pidx].wait()
            pbuf, _ = bufs[pidx % nb]
            _, poff, pout = sched[pidx]
            stores.append(
                pltpu.async_copy(pbuf, pout.at[pl.ds(base + poff, ch)], st_sem)
            )
        for st in stores:
            st.wait()

    return _gather2


_BS = 4096


def _mlp_block(x, y, w1_ref, b1, w2, b2, w3, b3, w4, b4):
    h = jnp.dot(x, w1_ref[0], preferred_element_type=jnp.float32)
    h = h + jnp.dot(y, w1_ref[1], preferred_element_type=jnp.float32)
    h = jnp.maximum(h + b1[...], 0.0)
    h = jnp.maximum(jnp.dot(h, w2[...], preferred_element_type=jnp.float32) + b2[...], 0.0)
    h = jnp.maximum(jnp.dot(h, w3[...], preferred_element_type=jnp.float32) + b3[...], 0.0)
    z = jax.lax.dot_general(
        w4[...], h, (((1,), (1,)), ((), ())),
        preferred_element_type=jnp.float32)[0] + b4[0]
    return 1.0 / (1.0 + jnp.exp(-z))


def _mlp_body(x_ref, y_ref, w1_ref, b1, w2, b2, w3, b3, w4, b4, o_ref):
    o_ref[...] = _mlp_block(x_ref[...], y_ref[...],
                            w1_ref, b1, w2, b2, w3, b3, w4, b4)


def _mlp_body_acc(prev_ref, x_ref, y_ref, w1_ref, b1, w2, b2, w3, b3, w4, b4,
                  o_ref):
    del prev_ref
    o_ref[...] = _mlp_block(x_ref[...], y_ref[...],
                            w1_ref, b1, w2, b2, w3, b3, w4, b4)


def _full(shape):
    return pl.BlockSpec(shape, lambda i: tuple(0 for _ in shape))


_W_SPECS = [
    _full((2, _D, 128)),
    _full((128,)),
    _full((128, 64)),
    _full((64,)),
    _full((64, 32)),
    _full((32,)),
    _full((1, 32)),
    _full((1,)),
]


def _mlp(blk_off, nblk, xg, yg, weights, prev=None):
    xy = pl.BlockSpec((_BS, _D), lambda i: (i, 0))
    out = pl.BlockSpec((_BS,), lambda i, o=blk_off: (o + i,))
    if prev is None:
        return pl.pallas_call(
            _mlp_body,
            grid=(nblk,),
            in_specs=[xy, xy] + _W_SPECS,
            out_specs=out,
            out_shape=jax.ShapeDtypeStruct((_B,), jnp.float32),
            compiler_params=pltpu.CompilerParams(
                dimension_semantics=("arbitrary",)),
        )(xg, yg, *weights)
    return pl.pallas_call(
        _mlp_body_acc,
        grid=(nblk,),
        in_specs=[pl.BlockSpec(memory_space=pl.ANY), xy, xy] + _W_SPECS,
        out_specs=out,
        out_shape=jax.ShapeDtypeStruct((_B,), jnp.float32),
        input_output_aliases={0: 0},
        compiler_params=pltpu.CompilerParams(
            dimension_semantics=("arbitrary",)),
    )(prev, xg, yg, *weights)


def kernel(user_input, item_input, user_emb, W1, b1, W2, b2, W3, b3, W4, b4):
    uidx = user_input.astype(jnp.int32)
    iidx = item_input.astype(jnp.int32)
    weights = (W1.reshape(2, _D, 128), b1, W2, b2, W3, b3, W4.reshape(1, 32), b4)
    out = None
    off = 0
    for bh in _SLICES:
        xg, yg = _make_gather(bh, off)(uidx, iidx, user_emb)
        out = _mlp(off // _BS, bh // _BS, xg, yg, weights, prev=out)
        off += bh
    return out
